# baseline (speedup 1.0000x reference)
import functools

import jax
import jax.numpy as jnp
from jax import lax
from jax.experimental import pallas as pl
from jax.experimental.pallas import tpu as pltpu
from jax.experimental.pallas import tpu_sc as plsc

B, S, D = 1, 2048, 768
E, K, DFF = 8, 2, 1024
T = B * S
LANES = 16
CB = 128


def _logits_kernel(x_ref, wr_ref, lt_ref):
    x = x_ref[...]
    lane = lax.broadcasted_iota(jnp.int32, (CB, E), 1)
    l = jnp.dot(x, wr_ref[...], preferred_element_type=jnp.float32)
    l = jnp.where(lane < E - 1, l, -1e30)
    lt_ref[...] = jnp.transpose(l)


def _topk_body(lt_hbm, wt_hbm, lt_v, wt_v):
    c = lax.axis_index("c")
    s = lax.axis_index("s")
    wid = s * 2 + c

    @pl.when(wid < T // CB)
    def _work():
        t0 = wid * CB
        pltpu.sync_copy(lt_hbm.at[:, pl.ds(t0, CB)], lt_v)
        for ci in range(CB // LANES):
            sl = pl.ds(ci * LANES, LANES)
            neg = jnp.full((LANES,), -1e30, jnp.float32)
            m1 = neg
            for e in range(E - 1):
                m1 = jnp.maximum(m1, lt_v[e, sl])
            idx1 = jnp.full((LANES,), 99.0, jnp.float32)
            for e in range(E - 2, -1, -1):
                le = lt_v[e, sl]
                idx1 = jnp.where(le == m1, float(e), idx1)
            m2 = neg
            for e in range(E - 1):
                le = lt_v[e, sl]
                m2 = jnp.maximum(m2, jnp.where(idx1 == float(e), neg, le))
            idx2 = jnp.full((LANES,), 99.0, jnp.float32)
            for e in range(E - 2, -1, -1):
                le = lt_v[e, sl]
                hit = jnp.logical_and(le == m2, idx1 != float(e))
                idx2 = jnp.where(hit, float(e), idx2)
            p2 = jnp.exp(m2 - m1)
            den = 1.0 + p2
            w1n = 1.0 / den
            w2n = p2 / den
            for e in range(E):
                row = jnp.where(idx1 == float(e - 1), w1n, 0.0)
                row = row + jnp.where(idx2 == float(e - 1), w2n, 0.0)
                if e == 0:
                    row = row + 1.0
                wt_v[e, sl] = row
        pltpu.sync_copy(wt_v, wt_hbm.at[:, pl.ds(t0, CB)])


def _moe_dense_kernel(x_ref, w1_ref, w2_ref, wt_ref, out_ref, w_ref):
    e = pl.program_id(0)

    @pl.when(e == 0)
    def _wt():
        w_ref[...] = jnp.transpose(wt_ref[...])

    x = x_ref[...]
    h = jnp.dot(x, w1_ref[0], preferred_element_type=jnp.float32)
    h = h * jax.lax.logistic(h)
    y = jnp.dot(h, w2_ref[0], preferred_element_type=jnp.float32)

    lane = lax.broadcasted_iota(jnp.int32, (T, E), 1)
    wcol = jnp.sum(jnp.where(lane == e, w_ref[...], 0.0), axis=1,
                   keepdims=True)
    contrib = y * wcol

    @pl.when(e == 0)
    def _init():
        out_ref[...] = contrib

    @pl.when(e > 0)
    def _acc():
        out_ref[...] += contrib


def kernel(hidden_states, Wr, W1, W2, interpret=False):
    x = hidden_states.reshape(T, D)
    wr_pad = jnp.zeros((D, E), jnp.float32).at[:, : E - 1].set(Wr)

    lt = pl.pallas_call(
        _logits_kernel,
        grid=(T // CB,),
        in_specs=[
            pl.BlockSpec((CB, D), lambda i: (i, 0)),
            pl.BlockSpec((D, E), lambda i: (0, 0)),
        ],
        out_specs=pl.BlockSpec((E, CB), lambda i: (0, i)),
        out_shape=jax.ShapeDtypeStruct((E, T), jnp.float32),
        interpret=interpret,
    )(x, wr_pad)

    mesh = plsc.VectorSubcoreMesh(core_axis_name="c", subcore_axis_name="s")
    wt = pl.kernel(
        _topk_body,
        mesh=mesh,
        out_type=jax.ShapeDtypeStruct((E, T), jnp.float32),
        scratch_types=[
            pltpu.VMEM((E, CB), jnp.float32),
            pltpu.VMEM((E, CB), jnp.float32),
        ],
    )(lt)

    out = pl.pallas_call(
        _moe_dense_kernel,
        grid=(E,),
        in_specs=[
            pl.BlockSpec((T, D), lambda e: (0, 0)),
            pl.BlockSpec((1, D, DFF), lambda e: (e, 0, 0)),
            pl.BlockSpec((1, DFF, D), lambda e: (e, 0, 0)),
            pl.BlockSpec((E, T), lambda e: (0, 0)),
        ],
        out_specs=pl.BlockSpec((T, D), lambda e: (0, 0)),
        out_shape=jax.ShapeDtypeStruct((T, D), jnp.float32),
        scratch_shapes=[pltpu.VMEM((T, E), jnp.float32)],
        interpret=interpret,
    )(x, W1, W2, wt)
    return out.reshape(B, S, D)


# speedup vs baseline: 1.0782x; 1.0782x over previous
import functools

import jax
import jax.numpy as jnp
from jax import lax
from jax.experimental import pallas as pl
from jax.experimental.pallas import tpu as pltpu
from jax.experimental.pallas import tpu_sc as plsc

B, S, D = 1, 2048, 768
E, K, DFF = 8, 2, 1024
T = B * S
LANES = 16
CB = 128


def _logits_kernel(x_ref, wr_ref, lt_ref):
    x = x_ref[...]
    lane = lax.broadcasted_iota(jnp.int32, (T, E), 1)
    l = jnp.dot(x, wr_ref[...], preferred_element_type=jnp.float32)
    l = jnp.where(lane < E - 1, l, -1e30)
    lt_ref[...] = jnp.transpose(l)


def _topk_body(lt_hbm, wt_hbm, lt_v, wt_v):
    c = lax.axis_index("c")
    s = lax.axis_index("s")
    wid = s * 2 + c

    @pl.when(wid < T // CB)
    def _work():
        t0 = wid * CB
        pltpu.sync_copy(lt_hbm.at[:, pl.ds(t0, CB)], lt_v)
        for ci in range(CB // LANES):
            sl = pl.ds(ci * LANES, LANES)
            neg = jnp.full((LANES,), -1e30, jnp.float32)
            m1 = neg
            for e in range(E - 1):
                m1 = jnp.maximum(m1, lt_v[e, sl])
            idx1 = jnp.full((LANES,), 99.0, jnp.float32)
            for e in range(E - 2, -1, -1):
                le = lt_v[e, sl]
                idx1 = jnp.where(le == m1, float(e), idx1)
            m2 = neg
            for e in range(E - 1):
                le = lt_v[e, sl]
                m2 = jnp.maximum(m2, jnp.where(idx1 == float(e), neg, le))
            idx2 = jnp.full((LANES,), 99.0, jnp.float32)
            for e in range(E - 2, -1, -1):
                le = lt_v[e, sl]
                hit = jnp.logical_and(le == m2, idx1 != float(e))
                idx2 = jnp.where(hit, float(e), idx2)
            p2 = jnp.exp(m2 - m1)
            den = 1.0 + p2
            w1n = 1.0 / den
            w2n = p2 / den
            for e in range(E):
                row = jnp.where(idx1 == float(e - 1), w1n, 0.0)
                row = row + jnp.where(idx2 == float(e - 1), w2n, 0.0)
                if e == 0:
                    row = row + 1.0
                wt_v[e, sl] = row
        pltpu.sync_copy(wt_v, wt_hbm.at[:, pl.ds(t0, CB)])


def _moe_dense_kernel(x_ref, w1_ref, w2_ref, wt_ref, out_ref, w_ref):
    e = pl.program_id(0)

    @pl.when(e == 0)
    def _wt():
        w_ref[...] = jnp.transpose(wt_ref[...])

    x = x_ref[...]
    h = jnp.dot(x, w1_ref[0], preferred_element_type=jnp.float32)
    h = h * jax.lax.logistic(h)
    y = jnp.dot(h, w2_ref[0], preferred_element_type=jnp.float32)

    lane = lax.broadcasted_iota(jnp.int32, (T, E), 1)
    wcol = jnp.sum(jnp.where(lane == e, w_ref[...], 0.0), axis=1,
                   keepdims=True)
    contrib = y * wcol

    @pl.when(e == 0)
    def _init():
        out_ref[...] = contrib

    @pl.when(e > 0)
    def _acc():
        out_ref[...] += contrib


def kernel(hidden_states, Wr, W1, W2, interpret=False):
    x = hidden_states.reshape(T, D)
    wr_pad = jnp.zeros((D, E), jnp.float32).at[:, : E - 1].set(Wr)

    lt = pl.pallas_call(
        _logits_kernel,
        grid=(1,),
        in_specs=[
            pl.BlockSpec((T, D), lambda i: (0, 0)),
            pl.BlockSpec((D, E), lambda i: (0, 0)),
        ],
        out_specs=pl.BlockSpec((E, T), lambda i: (0, 0)),
        out_shape=jax.ShapeDtypeStruct((E, T), jnp.float32),
        interpret=interpret,
    )(x, wr_pad)

    mesh = plsc.VectorSubcoreMesh(core_axis_name="c", subcore_axis_name="s")
    wt = pl.kernel(
        _topk_body,
        mesh=mesh,
        out_type=jax.ShapeDtypeStruct((E, T), jnp.float32),
        scratch_types=[
            pltpu.VMEM((E, CB), jnp.float32),
            pltpu.VMEM((E, CB), jnp.float32),
        ],
    )(lt)

    out = pl.pallas_call(
        _moe_dense_kernel,
        grid=(E,),
        in_specs=[
            pl.BlockSpec((T, D), lambda e: (0, 0)),
            pl.BlockSpec((1, D, DFF), lambda e: (e, 0, 0)),
            pl.BlockSpec((1, DFF, D), lambda e: (e, 0, 0)),
            pl.BlockSpec((E, T), lambda e: (0, 0)),
        ],
        out_specs=pl.BlockSpec((T, D), lambda e: (0, 0)),
        out_shape=jax.ShapeDtypeStruct((T, D), jnp.float32),
        scratch_shapes=[pltpu.VMEM((T, E), jnp.float32)],
        interpret=interpret,
    )(x, W1, W2, wt)
    return out.reshape(B, S, D)


# speedup vs baseline: 1.0876x; 1.0087x over previous
import functools

import jax
import jax.numpy as jnp
from jax import lax
from jax.experimental import pallas as pl
from jax.experimental.pallas import tpu as pltpu
from jax.experimental.pallas import tpu_sc as plsc

B, S, D = 1, 2048, 768
E, K, DFF = 8, 2, 1024
T = B * S
LANES = 16
CB = 128


def _logits_kernel(x_ref, wr_ref, lt_ref):
    x = x_ref[...]
    lane = lax.broadcasted_iota(jnp.int32, (T, E), 1)
    l = jnp.dot(x, wr_ref[...], preferred_element_type=jnp.float32)
    l = jnp.where(lane < E - 1, l, -1e30)
    lt_ref[...] = jnp.transpose(l)


def _topk_body(lt_hbm, wt_hbm, lt_v, wt_v):
    c = lax.axis_index("c")
    s = lax.axis_index("s")
    wid = s * 2 + c

    @pl.when(wid < T // CB)
    def _work():
        t0 = wid * CB
        pltpu.sync_copy(lt_hbm.at[:, pl.ds(t0, CB)], lt_v)
        for ci in range(CB // LANES):
            sl = pl.ds(ci * LANES, LANES)
            neg = jnp.full((LANES,), -1e30, jnp.float32)
            m1 = neg
            for e in range(E - 1):
                m1 = jnp.maximum(m1, lt_v[e, sl])
            idx1 = jnp.full((LANES,), 99.0, jnp.float32)
            for e in range(E - 2, -1, -1):
                le = lt_v[e, sl]
                idx1 = jnp.where(le == m1, float(e), idx1)
            m2 = neg
            for e in range(E - 1):
                le = lt_v[e, sl]
                m2 = jnp.maximum(m2, jnp.where(idx1 == float(e), neg, le))
            idx2 = jnp.full((LANES,), 99.0, jnp.float32)
            for e in range(E - 2, -1, -1):
                le = lt_v[e, sl]
                hit = jnp.logical_and(le == m2, idx1 != float(e))
                idx2 = jnp.where(hit, float(e), idx2)
            p2 = jnp.exp(m2 - m1)
            den = 1.0 + p2
            w1n = 1.0 / den
            w2n = p2 / den
            for e in range(E):
                row = jnp.where(idx1 == float(e - 1), w1n, 0.0)
                row = row + jnp.where(idx2 == float(e - 1), w2n, 0.0)
                if e == 0:
                    row = row + 1.0
                wt_v[e, sl] = row
        pltpu.sync_copy(wt_v, wt_hbm.at[:, pl.ds(t0, CB)])


def _moe_dense_kernel(x_ref, w1_ref, w2_ref, wt_ref, out_ref):
    e = pl.program_id(0)

    x = x_ref[...]
    h = jnp.dot(x, w1_ref[0], preferred_element_type=jnp.float32)
    h = h * jax.lax.logistic(h)
    y = jnp.dot(h, w2_ref[0], preferred_element_type=jnp.float32)

    w = jnp.transpose(wt_ref[...])
    lane = lax.broadcasted_iota(jnp.int32, (T, E), 1)
    wcol = jnp.sum(jnp.where(lane == e, w, 0.0), axis=1, keepdims=True)
    contrib = y * wcol

    @pl.when(e == 0)
    def _init():
        out_ref[...] = contrib

    @pl.when(e > 0)
    def _acc():
        out_ref[...] += contrib


def kernel(hidden_states, Wr, W1, W2, interpret=False):
    x = hidden_states.reshape(T, D)
    wr_pad = jnp.zeros((D, E), jnp.float32).at[:, : E - 1].set(Wr)

    lt = pl.pallas_call(
        _logits_kernel,
        grid=(1,),
        in_specs=[
            pl.BlockSpec((T, D), lambda i: (0, 0)),
            pl.BlockSpec((D, E), lambda i: (0, 0)),
        ],
        out_specs=pl.BlockSpec((E, T), lambda i: (0, 0)),
        out_shape=jax.ShapeDtypeStruct((E, T), jnp.float32),
        interpret=interpret,
    )(x, wr_pad)

    mesh = plsc.VectorSubcoreMesh(core_axis_name="c", subcore_axis_name="s")
    wt = pl.kernel(
        _topk_body,
        mesh=mesh,
        out_type=jax.ShapeDtypeStruct((E, T), jnp.float32),
        scratch_types=[
            pltpu.VMEM((E, CB), jnp.float32),
            pltpu.VMEM((E, CB), jnp.float32),
        ],
    )(lt)

    out = pl.pallas_call(
        _moe_dense_kernel,
        grid=(E,),
        in_specs=[
            pl.BlockSpec((T, D), lambda e: (0, 0)),
            pl.BlockSpec((1, D, DFF), lambda e: (e, 0, 0)),
            pl.BlockSpec((1, DFF, D), lambda e: (e, 0, 0)),
            pl.BlockSpec((E, T), lambda e: (0, 0)),
        ],
        out_specs=pl.BlockSpec((T, D), lambda e: (0, 0)),
        out_shape=jax.ShapeDtypeStruct((T, D), jnp.float32),
        interpret=interpret,
    )(x, W1, W2, wt)
    return out.reshape(B, S, D)
